# Initial kernel scaffold; baseline (speedup 1.0000x reference)
import jax
import jax.numpy as jnp
from jax.experimental import pallas as pl


def kernel(features, edge_index, W, attn_l, attn_r, bias, fc_W, fc_b):
    raise NotImplementedError("write your pallas kernel here")


# speedup vs baseline: 65.6521x; 65.6521x over previous
import functools

import jax
import jax.numpy as jnp
from jax import lax
from jax.experimental import pallas as pl
from jax.experimental.pallas import tpu as pltpu
from jax.experimental.pallas import tpu_sc as plsc

N_NODES = 10000
N_EDGES = 320000
IN_F = 128
N_HEADS = 4
FC_OUT = 10
HP = 12
GW = N_HEADS * HP
YW = 64

NC, NS, L = 2, 16, 16
NW = NC * NS
EW = N_EDGES // NW
WIN = 80
NWIN = EW // WIN
STRIPE = 624

BR = 400
GRID = N_NODES // BR


def _proj_body(x_ref, w_ref, al_ref, ar_ref, fcw_ref, y_ref, c_scr):
    @pl.when(pl.program_id(0) == 0)
    def _():
        cols = []
        for vec_ref in (al_ref, ar_ref):
            hs = []
            for h in range(N_HEADS):
                v = lax.dot_general(vec_ref[h], w_ref[h],
                                    (((0,), (0,)), ((), ())),
                                    preferred_element_type=jnp.float32)
                hs.append(v[:, None])
            cols.append(jnp.concatenate(hs, axis=1))
        pad2 = jnp.zeros((IN_F, HP - FC_OUT), jnp.float32)
        for h in range(N_HEADS):
            gh = lax.dot_general(w_ref[h], fcw_ref[...],
                                 (((0,), (1,)), ((), ())),
                                 preferred_element_type=jnp.float32)
            cols.append(jnp.concatenate([gh, pad2], axis=1))
        cols.append(jnp.zeros((IN_F, YW - 2 * N_HEADS - GW), jnp.float32))
        c_scr[...] = jnp.concatenate(cols, axis=1)

    y_ref[...] = jnp.dot(x_ref[...], c_scr[...],
                         preferred_element_type=jnp.float32)


def _project(x, W, attn_l, attn_r, fc_W):
    return pl.pallas_call(
        _proj_body,
        grid=(GRID,),
        in_specs=[
            pl.BlockSpec((BR, IN_F), lambda i: (i, 0)),
            pl.BlockSpec((N_HEADS, IN_F, IN_F), lambda i: (0, 0, 0)),
            pl.BlockSpec((N_HEADS, IN_F), lambda i: (0, 0)),
            pl.BlockSpec((N_HEADS, IN_F), lambda i: (0, 0)),
            pl.BlockSpec((FC_OUT, IN_F), lambda i: (0, 0)),
        ],
        out_specs=pl.BlockSpec((BR, YW), lambda i: (i, 0)),
        out_shape=jax.ShapeDtypeStruct((N_NODES, YW), jnp.float32),
        scratch_shapes=[pltpu.VMEM((IN_F, YW), jnp.float32)],
    )(x, W, attn_l, attn_r, fc_W)


@functools.lru_cache(maxsize=1)
def _build_edge_kernel():
    mesh = plsc.VectorSubcoreMesh(core_axis_name="c", subcore_axis_name="s",
                                  num_cores=NC, num_subcores=NS)
    return functools.partial(
        pl.kernel,
        out_type=jax.ShapeDtypeStruct((NC, N_NODES, GW), jnp.float32),
        mesh=mesh,
        compiler_params=pltpu.CompilerParams(needs_layout_passes=False, use_tc_tiling_on_sc=False),
        scratch_types=[
            pltpu.VMEM((N_NODES * N_HEADS,), jnp.float32),
            pltpu.VMEM((N_NODES * N_HEADS,), jnp.float32),
            pltpu.VMEM((WIN,), jnp.int32),
            pltpu.VMEM((WIN,), jnp.int32),
            pltpu.VMEM((WIN, GW), jnp.float32),
            pltpu.VMEM((WIN * N_HEADS,), jnp.float32),
            pltpu.VMEM_SHARED((N_NODES, GW), jnp.float32),
            pltpu.VMEM_SHARED((N_NODES * N_HEADS,), jnp.float32),
            pltpu.VMEM_SHARED((N_NODES * N_HEADS,), jnp.float32),
        ],
    )(_edge_body)


def _edge_body(src_hbm, dst_hbm, el_hbm, er_hbm, g_hbm, out_hbm,
               el_v, er_v, src_v, dst_v, gbuf, wbuf,
               out_sh, el_sh, er_sh):
    cid = lax.axis_index("c")
    sid = lax.axis_index("s")
    wid = cid * NS + sid

    lanes = lax.iota(jnp.int32, L)
    zero16 = jnp.zeros((L,), jnp.float32)

    def _zero_g(e, _):
        for p in range(GW // L):
            gbuf[e, pl.ds(p * L, L)] = zero16
        return 0

    lax.fori_loop(0, WIN, _zero_g, 0)

    r0 = sid * STRIPE
    for k in range(STRIPE // WIN):
        pltpu.sync_copy(gbuf, out_sh.at[pl.ds(r0 + k * WIN, WIN)])
    rem = STRIPE - (STRIPE // WIN) * WIN
    pltpu.sync_copy(gbuf.at[pl.ds(0, rem)],
                    out_sh.at[pl.ds(r0 + STRIPE - rem, rem)])
    tail0 = NS * STRIPE
    tail = N_NODES - tail0

    @pl.when(sid == NS - 1)
    def _():
        pltpu.sync_copy(gbuf.at[pl.ds(0, tail)], out_sh.at[pl.ds(tail0, tail)])

    @pl.when(sid == 0)
    def _():
        pltpu.sync_copy(el_hbm, el_sh)
        pltpu.sync_copy(er_hbm, er_sh)

    plsc.subcore_barrier()
    pltpu.sync_copy(el_sh, el_v)
    pltpu.sync_copy(er_sh, er_v)

    hmaps, dmasks = [], []
    for p in range(GW // L):
        c = lanes + L * p
        hmaps.append((c >= HP).astype(jnp.int32)
                     + (c >= 2 * HP).astype(jnp.int32)
                     + (c >= 3 * HP).astype(jnp.int32))
        dmasks.append(c % HP == FC_OUT)

    def _window(k, _):
        base = wid * EW + k * WIN
        pltpu.sync_copy(src_hbm.at[pl.ds(base, WIN)], src_v)
        pltpu.sync_copy(dst_hbm.at[pl.ds(base, WIN)], dst_v)
        pltpu.sync_copy(g_hbm.at[src_v], gbuf)

        def _wgrp(q, _):
            e4 = q * 4 + (lanes >> 2)
            h4 = lanes & 3
            srcr = plsc.load_gather(src_v, [e4])
            dstr = plsc.load_gather(dst_v, [e4])
            e = (plsc.load_gather(el_v, [srcr * N_HEADS + h4])
                 + plsc.load_gather(er_v, [dstr * N_HEADS + h4]))
            e = jnp.maximum(e, 0.2 * e)
            wbuf[pl.ds(q * L, L)] = jnp.exp(e)
            return 0

        lax.fori_loop(0, WIN * N_HEADS // L, _wgrp, 0)

        def _mul(e, _):
            for p in range(GW // L):
                wb = plsc.load_gather(wbuf, [e * N_HEADS + hmaps[p]])
                gv = gbuf[e, pl.ds(p * L, L)]
                gbuf[e, pl.ds(p * L, L)] = jnp.where(dmasks[p], wb, gv * wb)
            return 0

        lax.fori_loop(0, WIN, _mul, 0)


        pltpu.sync_copy(gbuf, out_sh.at[dst_v], add=True)
        return 0

    lax.fori_loop(0, NWIN, _window, 0)

    plsc.subcore_barrier()
    pltpu.sync_copy(out_sh.at[pl.ds(r0, STRIPE)],
                    out_hbm.at[cid, pl.ds(r0, STRIPE)])

    @pl.when(sid == NS - 1)
    def _():
        pltpu.sync_copy(out_sh.at[pl.ds(tail0, tail)],
                        out_hbm.at[cid, pl.ds(tail0, tail)])


def _combine_body(op_ref, bias_ref, fcw_ref, fcb_ref, o_ref):
    num = op_ref[0] + op_ref[1]
    const = lax.dot_general(bias_ref[...], fcw_ref[...],
                            (((1,), (1,)), ((), ())),
                            preferred_element_type=jnp.float32)
    const = const + fcb_ref[...]
    for h in range(N_HEADS):
        den = num[:, h * HP + FC_OUT:h * HP + FC_OUT + 1] + 1e-16
        o_ref[:, h * FC_OUT:(h + 1) * FC_OUT] = (
            num[:, h * HP:h * HP + FC_OUT] / den + const[h][None, :])


def _combine(out_p, bias, fc_W, fc_b):
    return pl.pallas_call(
        _combine_body,
        grid=(GRID,),
        in_specs=[
            pl.BlockSpec((NC, BR, GW), lambda i: (0, i, 0)),
            pl.BlockSpec((N_HEADS, IN_F), lambda i: (0, 0)),
            pl.BlockSpec((FC_OUT, IN_F), lambda i: (0, 0)),
            pl.BlockSpec((1, FC_OUT), lambda i: (0, 0)),
        ],
        out_specs=pl.BlockSpec((BR, N_HEADS * FC_OUT), lambda i: (i, 0)),
        out_shape=jax.ShapeDtypeStruct((N_NODES, N_HEADS * FC_OUT),
                                       jnp.float32),
    )(out_p, bias, fc_W, fc_b)


def kernel(features, edge_index, W, attn_l, attn_r, bias, fc_W, fc_b):
    src = edge_index[0].astype(jnp.int32)
    dst = edge_index[1].astype(jnp.int32)

    y = _project(features, W, attn_l, attn_r, fc_W)
    el = y[:, 0:N_HEADS].reshape(-1)
    er = y[:, N_HEADS:2 * N_HEADS].reshape(-1)
    g = y[:, 2 * N_HEADS:2 * N_HEADS + GW]

    out_p = _build_edge_kernel()(src, dst, el, er, g)
    out = _combine(out_p, bias, fc_W, fc_b.reshape(1, FC_OUT))
    return out.reshape(N_NODES, N_HEADS, FC_OUT)


# speedup vs baseline: 162.4627x; 2.4746x over previous
import functools

import jax
import jax.numpy as jnp
from jax import lax
from jax.experimental import pallas as pl
from jax.experimental.pallas import tpu as pltpu
from jax.experimental.pallas import tpu_sc as plsc

N_NODES = 10000
N_EDGES = 320000
IN_F = 128
N_HEADS = 4
FC_OUT = 10
HP = 12
GW = N_HEADS * HP
YW = 64

NC, NS, L = 2, 16, 16
NW = NC * NS
EW = N_EDGES // NW
WIN = 80
NWIN = EW // WIN
STRIPE = 624

BR = 400
GRID = N_NODES // BR


def _proj_body(x_ref, w_ref, al_ref, ar_ref, fcw_ref,
               el_ref, er_ref, g_ref, c_scr):
    @pl.when(pl.program_id(0) == 0)
    def _():
        cols = []
        for vec_ref in (al_ref, ar_ref):
            hs = []
            for h in range(N_HEADS):
                v = lax.dot_general(vec_ref[h], w_ref[h],
                                    (((0,), (0,)), ((), ())),
                                    preferred_element_type=jnp.float32)
                hs.append(v[:, None])
            cols.append(jnp.concatenate(hs, axis=1))
        pad2 = jnp.zeros((IN_F, HP - FC_OUT), jnp.float32)
        for h in range(N_HEADS):
            gh = lax.dot_general(w_ref[h], fcw_ref[...],
                                 (((0,), (1,)), ((), ())),
                                 preferred_element_type=jnp.float32)
            cols.append(jnp.concatenate([gh, pad2], axis=1))
        c_scr[...] = jnp.concatenate(cols, axis=1)

    y = jnp.dot(x_ref[...], c_scr[...], preferred_element_type=jnp.float32)
    el_ref[...] = y[:, 0:N_HEADS]
    er_ref[...] = y[:, N_HEADS:2 * N_HEADS]
    cg = 2 * N_HEADS
    colid = lax.broadcasted_iota(jnp.int32, (1, GW), 1)
    g_ref[...] = jnp.where(colid % HP == FC_OUT, 1.0, y[:, cg:cg + GW])


def _project(x, W, attn_l, attn_r, fc_W):
    return pl.pallas_call(
        _proj_body,
        grid=(GRID,),
        in_specs=[
            pl.BlockSpec((BR, IN_F), lambda i: (i, 0)),
            pl.BlockSpec((N_HEADS, IN_F, IN_F), lambda i: (0, 0, 0)),
            pl.BlockSpec((N_HEADS, IN_F), lambda i: (0, 0)),
            pl.BlockSpec((N_HEADS, IN_F), lambda i: (0, 0)),
            pl.BlockSpec((FC_OUT, IN_F), lambda i: (0, 0)),
        ],
        out_specs=[
            pl.BlockSpec((BR, N_HEADS), lambda i: (i, 0)),
            pl.BlockSpec((BR, N_HEADS), lambda i: (i, 0)),
            pl.BlockSpec((BR, GW), lambda i: (i, 0)),
        ],
        out_shape=[
            jax.ShapeDtypeStruct((N_NODES, N_HEADS), jnp.float32),
            jax.ShapeDtypeStruct((N_NODES, N_HEADS), jnp.float32),
            jax.ShapeDtypeStruct((N_NODES, GW), jnp.float32),
        ],
        scratch_shapes=[pltpu.VMEM((IN_F, YW - 2 * N_HEADS), jnp.float32)],
    )(x, W, attn_l, attn_r, fc_W)


@functools.lru_cache(maxsize=1)
def _build_edge_kernel():
    mesh = plsc.VectorSubcoreMesh(core_axis_name="c", subcore_axis_name="s",
                                  num_cores=NC, num_subcores=NS)
    return functools.partial(
        pl.kernel,
        out_type=jax.ShapeDtypeStruct((NC, N_NODES, GW), jnp.float32),
        mesh=mesh,
        compiler_params=pltpu.CompilerParams(needs_layout_passes=False,
                                             use_tc_tiling_on_sc=False),
        scratch_types=[
            pltpu.VMEM((N_NODES * N_HEADS,), jnp.float32),
            pltpu.VMEM((N_NODES * N_HEADS,), jnp.float32),
            pltpu.VMEM((WIN,), jnp.int32),
            pltpu.VMEM((WIN,), jnp.int32),
            pltpu.VMEM((WIN,), jnp.int32),
            pltpu.VMEM((WIN,), jnp.int32),
            pltpu.VMEM((WIN,), jnp.int32),
            pltpu.VMEM((WIN,), jnp.int32),
            pltpu.VMEM((WIN, GW), jnp.float32),
            pltpu.VMEM((WIN, GW), jnp.float32),
            pltpu.VMEM((WIN * N_HEADS,), jnp.float32),
            pltpu.VMEM_SHARED((N_NODES, GW), jnp.float32),
            pltpu.VMEM_SHARED((N_NODES * N_HEADS,), jnp.float32),
            pltpu.VMEM_SHARED((N_NODES * N_HEADS,), jnp.float32),
            pltpu.SemaphoreType.DMA,
            pltpu.SemaphoreType.DMA,
            pltpu.SemaphoreType.DMA,
            pltpu.SemaphoreType.DMA,
            pltpu.SemaphoreType.DMA,
            pltpu.SemaphoreType.DMA,
        ],
    )(_edge_body)


def _vgather(vec, idx):
    dn = lax.GatherDimensionNumbers(offset_dims=(), collapsed_slice_dims=(0,),
                                    start_index_map=(0,))
    return lax.gather(vec, idx[:, None], dn, slice_sizes=(1,),
                      mode=lax.GatherScatterMode.PROMISE_IN_BOUNDS)


def _edge_body(ei_hbm, el_hbm, er_hbm, g_hbm, out_hbm,
               el_v, er_v, src0, src1, dst0, dst1, sci0, sci1,
               gbuf0, gbuf1, wbuf,
               out_sh, el_sh, er_sh,
               sem_sd0, sem_sd1, sem_g0, sem_g1, sem_sc0, sem_sc1):
    cid = lax.axis_index("c")
    sid = lax.axis_index("s")
    wid = cid * NS + sid

    srcs, dsts, scis = (src0, src1), (dst0, dst1), (sci0, sci1)
    gbufs = (gbuf0, gbuf1)
    sem_sd, sem_g, sem_sc = (sem_sd0, sem_sd1), (sem_g0, sem_g1), (sem_sc0,
                                                                   sem_sc1)

    lanes = lax.iota(jnp.int32, L)
    zero16 = jnp.zeros((L,), jnp.float32)

    def _zero_g(e, _):
        for pch in range(GW // L):
            gbuf0[e, pl.ds(pch * L, L)] = zero16
        return 0

    lax.fori_loop(0, WIN, _zero_g, 0)

    r0 = sid * STRIPE
    for k in range(STRIPE // WIN):
        pltpu.sync_copy(gbuf0, out_sh.at[pl.ds(r0 + k * WIN, WIN)])
    rem = STRIPE - (STRIPE // WIN) * WIN
    pltpu.sync_copy(gbuf0.at[pl.ds(0, rem)],
                    out_sh.at[pl.ds(r0 + STRIPE - rem, rem)])
    tail0 = NS * STRIPE
    tail = N_NODES - tail0

    @pl.when(sid == NS - 1)
    def _():
        pltpu.sync_copy(gbuf0.at[pl.ds(0, tail)], out_sh.at[pl.ds(tail0, tail)])

    @pl.when(sid == 0)
    def _():
        pltpu.sync_copy(el_hbm, el_sh)

    @pl.when(sid == 1)
    def _():
        pltpu.sync_copy(er_hbm, er_sh)

    plsc.subcore_barrier()
    pltpu.sync_copy(el_sh, el_v)
    pltpu.sync_copy(er_sh, er_v)

    hmaps = []
    for pch in range(GW // L):
        c = lanes + L * pch
        hmaps.append((c >= HP).astype(jnp.int32)
                     + (c >= 2 * HP).astype(jnp.int32)
                     + (c >= 3 * HP).astype(jnp.int32))

    def start_sd(w, ph):
        base = wid * EW + w * WIN
        pltpu.async_copy(ei_hbm.at[0, pl.ds(base, WIN)], srcs[ph], sem_sd[ph])
        pltpu.async_copy(ei_hbm.at[1, pl.ds(base, WIN)], dsts[ph], sem_sd[ph])

    def wait_sd(ph):
        pltpu.make_async_copy(ei_hbm.at[0, pl.ds(0, WIN)], srcs[ph],
                              sem_sd[ph]).wait()
        pltpu.make_async_copy(ei_hbm.at[1, pl.ds(0, WIN)], dsts[ph],
                              sem_sd[ph]).wait()

    def compute_w(ph):
        def _wgrp(q, _):
            e4 = q * 4 + (lanes >> 2)
            h4 = lanes & 3
            srcr = plsc.load_gather(srcs[ph], [e4])
            dstr = plsc.load_gather(dsts[ph], [e4])
            e = (plsc.load_gather(el_v, [srcr * N_HEADS + h4])
                 + plsc.load_gather(er_v, [dstr * N_HEADS + h4]))
            e = jnp.maximum(e, 0.2 * e)
            wbuf[pl.ds(q * L, L)] = jnp.exp(e)
            return 0

        lax.fori_loop(0, WIN * N_HEADS // L, _wgrp, 0)

    def copy_sci(ph):
        def _cp(q, _):
            scis[ph][pl.ds(q * L, L)] = dsts[ph][pl.ds(q * L, L)]
            return 0

        lax.fori_loop(0, WIN // L, _cp, 0)

    def mul(ph):
        def _mgrp(q, _):
            w16 = wbuf[pl.ds(q * L, L)]
            for j in range(4):
                e = q * 4 + j
                for pch in range(GW // L):
                    wb = _vgather(w16, hmaps[pch] + 4 * j)
                    gv = gbufs[ph][e, pl.ds(pch * L, L)]
                    gbufs[ph][e, pl.ds(pch * L, L)] = gv * wb
            return 0

        lax.fori_loop(0, WIN * N_HEADS // L, _mgrp, 0)

    def phase(w, ph):
        wait_sd(ph)

        @pl.when(w >= 2)
        def _():
            pltpu.make_async_copy(gbufs[ph], out_sh.at[scis[ph]],
                                  sem_sc[ph]).wait()

        gdesc = pltpu.async_copy(g_hbm.at[srcs[ph]], gbufs[ph], sem_g[ph])
        compute_w(ph)
        copy_sci(ph)
        gdesc.wait()

        @pl.when(w + 2 < NWIN)
        def _():
            start_sd(w + 2, ph)

        mul(ph)
        pltpu.async_copy(gbufs[ph], out_sh.at[scis[ph]], sem_sc[ph], add=True)

    start_sd(jnp.int32(0), 0)
    start_sd(jnp.int32(1), 1)

    def _pair(i, _):
        w = i * 2
        phase(w, 0)

        @pl.when(w + 1 < NWIN)
        def _():
            phase(w + 1, 1)

        return 0

    lax.fori_loop(0, (NWIN + 1) // 2, _pair, 0)

    for ph in range(2):
        pltpu.make_async_copy(gbufs[ph], out_sh.at[scis[ph]],
                              sem_sc[ph]).wait()

    plsc.subcore_barrier()
    pltpu.sync_copy(out_sh.at[pl.ds(r0, STRIPE)],
                    out_hbm.at[cid, pl.ds(r0, STRIPE)])

    @pl.when(sid == NS - 1)
    def _():
        pltpu.sync_copy(out_sh.at[pl.ds(tail0, tail)],
                        out_hbm.at[cid, pl.ds(tail0, tail)])


def _combine_body(op_ref, bias_ref, fcw_ref, fcb_ref, o_ref):
    num = op_ref[0] + op_ref[1]
    const = lax.dot_general(bias_ref[...], fcw_ref[...],
                            (((1,), (1,)), ((), ())),
                            preferred_element_type=jnp.float32)
    const = const + fcb_ref[...]
    for h in range(N_HEADS):
        den = num[:, h * HP + FC_OUT:h * HP + FC_OUT + 1] + 1e-16
        o_ref[:, h * FC_OUT:(h + 1) * FC_OUT] = (
            num[:, h * HP:h * HP + FC_OUT] / den + const[h][None, :])


def _combine(out_p, bias, fc_W, fc_b):
    return pl.pallas_call(
        _combine_body,
        grid=(GRID,),
        in_specs=[
            pl.BlockSpec((NC, BR, GW), lambda i: (0, i, 0)),
            pl.BlockSpec((N_HEADS, IN_F), lambda i: (0, 0)),
            pl.BlockSpec((FC_OUT, IN_F), lambda i: (0, 0)),
            pl.BlockSpec((1, FC_OUT), lambda i: (0, 0)),
        ],
        out_specs=pl.BlockSpec((BR, N_HEADS * FC_OUT), lambda i: (i, 0)),
        out_shape=jax.ShapeDtypeStruct((N_NODES, N_HEADS * FC_OUT),
                                       jnp.float32),
    )(out_p, bias, fc_W, fc_b)


def kernel(features, edge_index, W, attn_l, attn_r, bias, fc_W, fc_b):
    ei = edge_index.astype(jnp.int32)
    el, er, g = _project(features, W, attn_l, attn_r, fc_W)
    out_p = _build_edge_kernel()(ei, el.reshape(-1), er.reshape(-1), g)
    out = _combine(out_p, bias, fc_W, fc_b.reshape(1, FC_OUT))
    return out.reshape(N_NODES, N_HEADS, FC_OUT)


# speedup vs baseline: 169.3225x; 1.0422x over previous
import functools

import jax
import jax.numpy as jnp
from jax import lax
from jax.experimental import pallas as pl
from jax.experimental.pallas import tpu as pltpu
from jax.experimental.pallas import tpu_sc as plsc

N_NODES = 10000
N_EDGES = 320000
IN_F = 128
N_HEADS = 4
FC_OUT = 10
HP = 12
GW = N_HEADS * HP
YW = 64

NC, NS, L = 2, 16, 16
NW = NC * NS
EW = N_EDGES // NW
WIN = 80
NWIN = EW // WIN
STRIPE = 624

BR = 400
GRID = N_NODES // BR


def _proj_body(x_ref, w_ref, al_ref, ar_ref, fcw_ref,
               el_ref, er_ref, g_ref, c_scr):
    @pl.when(pl.program_id(0) == 0)
    def _():
        cols = []
        for vec_ref in (al_ref, ar_ref):
            hs = []
            for h in range(N_HEADS):
                v = lax.dot_general(vec_ref[h], w_ref[h],
                                    (((0,), (0,)), ((), ())),
                                    preferred_element_type=jnp.float32)
                hs.append(v[:, None])
            cols.append(jnp.concatenate(hs, axis=1))
        pad2 = jnp.zeros((IN_F, HP - FC_OUT), jnp.float32)
        for h in range(N_HEADS):
            gh = lax.dot_general(w_ref[h], fcw_ref[...],
                                 (((0,), (1,)), ((), ())),
                                 preferred_element_type=jnp.float32)
            cols.append(jnp.concatenate([gh, pad2], axis=1))
        c_scr[...] = jnp.concatenate(cols, axis=1)

    y = jnp.dot(x_ref[...], c_scr[...], preferred_element_type=jnp.float32)
    el_ref[...] = y[:, 0:N_HEADS]
    er_ref[...] = y[:, N_HEADS:2 * N_HEADS]
    cg = 2 * N_HEADS
    colid = lax.broadcasted_iota(jnp.int32, (1, GW), 1)
    g_ref[...] = jnp.where(colid % HP == FC_OUT, 1.0, y[:, cg:cg + GW])


def _project(x, W, attn_l, attn_r, fc_W):
    return pl.pallas_call(
        _proj_body,
        grid=(GRID,),
        in_specs=[
            pl.BlockSpec((BR, IN_F), lambda i: (i, 0)),
            pl.BlockSpec((N_HEADS, IN_F, IN_F), lambda i: (0, 0, 0)),
            pl.BlockSpec((N_HEADS, IN_F), lambda i: (0, 0)),
            pl.BlockSpec((N_HEADS, IN_F), lambda i: (0, 0)),
            pl.BlockSpec((FC_OUT, IN_F), lambda i: (0, 0)),
        ],
        out_specs=[
            pl.BlockSpec((BR, N_HEADS), lambda i: (i, 0)),
            pl.BlockSpec((BR, N_HEADS), lambda i: (i, 0)),
            pl.BlockSpec((BR, GW), lambda i: (i, 0)),
        ],
        out_shape=[
            jax.ShapeDtypeStruct((N_NODES, N_HEADS), jnp.float32),
            jax.ShapeDtypeStruct((N_NODES, N_HEADS), jnp.float32),
            jax.ShapeDtypeStruct((N_NODES, GW), jnp.float32),
        ],
        scratch_shapes=[pltpu.VMEM((IN_F, YW - 2 * N_HEADS), jnp.float32)],
    )(x, W, attn_l, attn_r, fc_W)


@functools.lru_cache(maxsize=1)
def _build_edge_kernel():
    mesh = plsc.VectorSubcoreMesh(core_axis_name="c", subcore_axis_name="s",
                                  num_cores=NC, num_subcores=NS)
    return functools.partial(
        pl.kernel,
        out_type=jax.ShapeDtypeStruct((NC, N_NODES, GW), jnp.float32),
        mesh=mesh,
        compiler_params=pltpu.CompilerParams(needs_layout_passes=False,
                                             use_tc_tiling_on_sc=False),
        scratch_types=[
            pltpu.VMEM((N_NODES * N_HEADS,), jnp.float32),
            pltpu.VMEM((N_NODES * N_HEADS,), jnp.float32),
            pltpu.VMEM((WIN,), jnp.int32),
            pltpu.VMEM((WIN,), jnp.int32),
            pltpu.VMEM((WIN,), jnp.int32),
            pltpu.VMEM((WIN,), jnp.int32),
            pltpu.VMEM((WIN,), jnp.int32),
            pltpu.VMEM((WIN,), jnp.int32),
            pltpu.VMEM((WIN, GW), jnp.float32),
            pltpu.VMEM((WIN, GW), jnp.float32),
            pltpu.VMEM((WIN * N_HEADS,), jnp.float32),
            pltpu.VMEM_SHARED((N_NODES, GW), jnp.float32),
            pltpu.VMEM_SHARED((N_NODES * N_HEADS,), jnp.float32),
            pltpu.VMEM_SHARED((N_NODES * N_HEADS,), jnp.float32),
            pltpu.SemaphoreType.DMA,
            pltpu.SemaphoreType.DMA,
            pltpu.SemaphoreType.DMA,
            pltpu.SemaphoreType.DMA,
            pltpu.SemaphoreType.DMA,
            pltpu.SemaphoreType.DMA,
        ],
    )(_edge_body)


def _vgather(vec, idx):
    dn = lax.GatherDimensionNumbers(offset_dims=(), collapsed_slice_dims=(0,),
                                    start_index_map=(0,))
    return lax.gather(vec, idx[:, None], dn, slice_sizes=(1,),
                      mode=lax.GatherScatterMode.PROMISE_IN_BOUNDS)


def _edge_body(ei_hbm, el_hbm, er_hbm, g_hbm, out_hbm,
               el_v, er_v, src0, src1, dst0, dst1, sci0, sci1,
               gbuf0, gbuf1, wbuf,
               out_sh, el_sh, er_sh,
               sem_sd0, sem_sd1, sem_g0, sem_g1, sem_sc0, sem_sc1):
    cid = lax.axis_index("c")
    sid = lax.axis_index("s")
    wid = cid * NS + sid

    srcs, dsts, scis = (src0, src1), (dst0, dst1), (sci0, sci1)
    gbufs = (gbuf0, gbuf1)
    sem_sd, sem_g, sem_sc = (sem_sd0, sem_sd1), (sem_g0, sem_g1), (sem_sc0,
                                                                   sem_sc1)

    lanes = lax.iota(jnp.int32, L)
    zero16 = jnp.zeros((L,), jnp.float32)

    def _zero_g(e, _):
        for pch in range(GW // L):
            gbuf0[e, pl.ds(pch * L, L)] = zero16
        return 0

    lax.fori_loop(0, WIN, _zero_g, 0)

    r0 = sid * STRIPE
    for k in range(STRIPE // WIN):
        pltpu.sync_copy(gbuf0, out_sh.at[pl.ds(r0 + k * WIN, WIN)])
    rem = STRIPE - (STRIPE // WIN) * WIN
    pltpu.sync_copy(gbuf0.at[pl.ds(0, rem)],
                    out_sh.at[pl.ds(r0 + STRIPE - rem, rem)])
    tail0 = NS * STRIPE
    tail = N_NODES - tail0

    @pl.when(sid == NS - 1)
    def _():
        pltpu.sync_copy(gbuf0.at[pl.ds(0, tail)], out_sh.at[pl.ds(tail0, tail)])

    @pl.when(sid == 0)
    def _():
        pltpu.sync_copy(el_hbm, el_sh)

    @pl.when(sid == 1)
    def _():
        pltpu.sync_copy(er_hbm, er_sh)

    plsc.subcore_barrier()
    pltpu.sync_copy(el_sh, el_v)
    pltpu.sync_copy(er_sh, er_v)

    hmaps = []
    for pch in range(GW // L):
        c = lanes + L * pch
        hmaps.append((c >= HP).astype(jnp.int32)
                     + (c >= 2 * HP).astype(jnp.int32)
                     + (c >= 3 * HP).astype(jnp.int32))

    def start_sd(w, ph):
        base = wid * EW + w * WIN
        pltpu.async_copy(ei_hbm.at[0, pl.ds(base, WIN)], srcs[ph], sem_sd[ph])
        pltpu.async_copy(ei_hbm.at[1, pl.ds(base, WIN)], dsts[ph], sem_sd[ph])

    def wait_sd(ph):
        pltpu.make_async_copy(ei_hbm.at[0, pl.ds(0, WIN)], srcs[ph],
                              sem_sd[ph]).wait()
        pltpu.make_async_copy(ei_hbm.at[1, pl.ds(0, WIN)], dsts[ph],
                              sem_sd[ph]).wait()

    def compute_w(ph):
        h4 = lanes & 3
        e4base = lanes >> 2
        for q in range(WIN * N_HEADS // L):
            e4 = e4base + q * 4
            srcr = plsc.load_gather(srcs[ph], [e4])
            dstr = plsc.load_gather(dsts[ph], [e4])
            e = (plsc.load_gather(el_v, [srcr * N_HEADS + h4])
                 + plsc.load_gather(er_v, [dstr * N_HEADS + h4]))
            e = jnp.maximum(e, 0.2 * e)
            wbuf[pl.ds(q * L, L)] = jnp.exp(e)

    def copy_sci(ph):
        for q in range(WIN // L):
            scis[ph][pl.ds(q * L, L)] = dsts[ph][pl.ds(q * L, L)]

    def mul(ph):
        for q in range(WIN * N_HEADS // L):
            w16 = wbuf[pl.ds(q * L, L)]
            for j in range(4):
                e = q * 4 + j
                for pch in range(GW // L):
                    wb = _vgather(w16, hmaps[pch] + 4 * j)
                    gv = gbufs[ph][e, pl.ds(pch * L, L)]
                    gbufs[ph][e, pl.ds(pch * L, L)] = gv * wb

    def phase(w, ph):
        wait_sd(ph)

        @pl.when(w >= 2)
        def _():
            pltpu.make_async_copy(gbufs[ph], out_sh.at[scis[ph]],
                                  sem_sc[ph]).wait()

        gdesc = pltpu.async_copy(g_hbm.at[srcs[ph]], gbufs[ph], sem_g[ph])
        compute_w(ph)
        copy_sci(ph)
        gdesc.wait()

        @pl.when(w + 2 < NWIN)
        def _():
            start_sd(w + 2, ph)

        mul(ph)
        pltpu.async_copy(gbufs[ph], out_sh.at[scis[ph]], sem_sc[ph], add=True)

    start_sd(jnp.int32(0), 0)
    start_sd(jnp.int32(1), 1)

    def _pair(i, _):
        w = i * 2
        phase(w, 0)

        @pl.when(w + 1 < NWIN)
        def _():
            phase(w + 1, 1)

        return 0

    lax.fori_loop(0, (NWIN + 1) // 2, _pair, 0)

    for ph in range(2):
        pltpu.make_async_copy(gbufs[ph], out_sh.at[scis[ph]],
                              sem_sc[ph]).wait()

    plsc.subcore_barrier()
    pltpu.sync_copy(out_sh.at[pl.ds(r0, STRIPE)],
                    out_hbm.at[cid, pl.ds(r0, STRIPE)])

    @pl.when(sid == NS - 1)
    def _():
        pltpu.sync_copy(out_sh.at[pl.ds(tail0, tail)],
                        out_hbm.at[cid, pl.ds(tail0, tail)])


def _combine_body(op_ref, bias_ref, fcw_ref, fcb_ref, o_ref):
    num = op_ref[0] + op_ref[1]
    const = lax.dot_general(bias_ref[...], fcw_ref[...],
                            (((1,), (1,)), ((), ())),
                            preferred_element_type=jnp.float32)
    const = const + fcb_ref[...]
    for h in range(N_HEADS):
        den = num[:, h * HP + FC_OUT:h * HP + FC_OUT + 1] + 1e-16
        o_ref[:, h * FC_OUT:(h + 1) * FC_OUT] = (
            num[:, h * HP:h * HP + FC_OUT] / den + const[h][None, :])


def _combine(out_p, bias, fc_W, fc_b):
    return pl.pallas_call(
        _combine_body,
        grid=(GRID,),
        in_specs=[
            pl.BlockSpec((NC, BR, GW), lambda i: (0, i, 0)),
            pl.BlockSpec((N_HEADS, IN_F), lambda i: (0, 0)),
            pl.BlockSpec((FC_OUT, IN_F), lambda i: (0, 0)),
            pl.BlockSpec((1, FC_OUT), lambda i: (0, 0)),
        ],
        out_specs=pl.BlockSpec((BR, N_HEADS * FC_OUT), lambda i: (i, 0)),
        out_shape=jax.ShapeDtypeStruct((N_NODES, N_HEADS * FC_OUT),
                                       jnp.float32),
    )(out_p, bias, fc_W, fc_b)


def kernel(features, edge_index, W, attn_l, attn_r, bias, fc_W, fc_b):
    ei = edge_index.astype(jnp.int32)
    el, er, g = _project(features, W, attn_l, attn_r, fc_W)
    out_p = _build_edge_kernel()(ei, el.reshape(-1), er.reshape(-1), g)
    out = _combine(out_p, bias, fc_W, fc_b.reshape(1, FC_OUT))
    return out.reshape(N_NODES, N_HEADS, FC_OUT)


# speedup vs baseline: 185.4028x; 1.0950x over previous
import functools

import jax
import jax.numpy as jnp
from jax import lax
from jax.experimental import pallas as pl
from jax.experimental.pallas import tpu as pltpu
from jax.experimental.pallas import tpu_sc as plsc

N_NODES = 10000
N_EDGES = 320000
IN_F = 128
N_HEADS = 4
FC_OUT = 10
HP = 12
GW = N_HEADS * HP
YW = 64

NC, NS, L = 2, 16, 16
NW = NC * NS
EW = N_EDGES // NW
WIN = 80
NWIN = EW // WIN
STRIPE = 624

BR = 2000
GRID = N_NODES // BR


def _proj_body(x_ref, w_ref, al_ref, ar_ref, fcw_ref,
               el_ref, er_ref, g_ref, c_scr):
    @pl.when(pl.program_id(0) == 0)
    def _():
        cols = []
        for vec_ref in (al_ref, ar_ref):
            hs = []
            for h in range(N_HEADS):
                v = lax.dot_general(vec_ref[h], w_ref[h],
                                    (((0,), (0,)), ((), ())),
                                    preferred_element_type=jnp.float32)
                hs.append(v[:, None])
            cols.append(jnp.concatenate(hs, axis=1))
        pad2 = jnp.zeros((IN_F, HP - FC_OUT), jnp.float32)
        for h in range(N_HEADS):
            gh = lax.dot_general(w_ref[h], fcw_ref[...],
                                 (((0,), (1,)), ((), ())),
                                 preferred_element_type=jnp.float32)
            cols.append(jnp.concatenate([gh, pad2], axis=1))
        c_scr[...] = jnp.concatenate(cols, axis=1)

    y = jnp.dot(x_ref[...], c_scr[...], preferred_element_type=jnp.float32)
    el_ref[...] = y[:, 0:N_HEADS]
    er_ref[...] = y[:, N_HEADS:2 * N_HEADS]
    cg = 2 * N_HEADS
    colid = lax.broadcasted_iota(jnp.int32, (1, GW), 1)
    g_ref[...] = jnp.where(colid % HP == FC_OUT, 1.0, y[:, cg:cg + GW])


def _project(x, W, attn_l, attn_r, fc_W):
    return pl.pallas_call(
        _proj_body,
        grid=(GRID,),
        in_specs=[
            pl.BlockSpec((BR, IN_F), lambda i: (i, 0)),
            pl.BlockSpec((N_HEADS, IN_F, IN_F), lambda i: (0, 0, 0)),
            pl.BlockSpec((N_HEADS, IN_F), lambda i: (0, 0)),
            pl.BlockSpec((N_HEADS, IN_F), lambda i: (0, 0)),
            pl.BlockSpec((FC_OUT, IN_F), lambda i: (0, 0)),
        ],
        out_specs=[
            pl.BlockSpec((BR, N_HEADS), lambda i: (i, 0)),
            pl.BlockSpec((BR, N_HEADS), lambda i: (i, 0)),
            pl.BlockSpec((BR, GW), lambda i: (i, 0)),
        ],
        out_shape=[
            jax.ShapeDtypeStruct((N_NODES, N_HEADS), jnp.float32),
            jax.ShapeDtypeStruct((N_NODES, N_HEADS), jnp.float32),
            jax.ShapeDtypeStruct((N_NODES, GW), jnp.float32),
        ],
        scratch_shapes=[pltpu.VMEM((IN_F, YW - 2 * N_HEADS), jnp.float32)],
    )(x, W, attn_l, attn_r, fc_W)


@functools.lru_cache(maxsize=1)
def _build_edge_kernel():
    mesh = plsc.VectorSubcoreMesh(core_axis_name="c", subcore_axis_name="s",
                                  num_cores=NC, num_subcores=NS)
    return functools.partial(
        pl.kernel,
        out_type=jax.ShapeDtypeStruct((NC, N_NODES, GW), jnp.float32),
        mesh=mesh,
        compiler_params=pltpu.CompilerParams(needs_layout_passes=False,
                                             use_tc_tiling_on_sc=False),
        scratch_types=[
            pltpu.VMEM((N_NODES * N_HEADS,), jnp.float32),
            pltpu.VMEM((N_NODES * N_HEADS,), jnp.float32),
            pltpu.VMEM((WIN,), jnp.int32),
            pltpu.VMEM((WIN,), jnp.int32),
            pltpu.VMEM((WIN,), jnp.int32),
            pltpu.VMEM((WIN,), jnp.int32),
            pltpu.VMEM((WIN,), jnp.int32),
            pltpu.VMEM((WIN,), jnp.int32),
            pltpu.VMEM((WIN, GW), jnp.float32),
            pltpu.VMEM((WIN, GW), jnp.float32),
            pltpu.VMEM((WIN * N_HEADS,), jnp.float32),
            pltpu.VMEM_SHARED((N_NODES, GW), jnp.float32),
            pltpu.VMEM_SHARED((N_NODES * N_HEADS,), jnp.float32),
            pltpu.VMEM_SHARED((N_NODES * N_HEADS,), jnp.float32),
            pltpu.SemaphoreType.DMA,
            pltpu.SemaphoreType.DMA,
            pltpu.SemaphoreType.DMA,
            pltpu.SemaphoreType.DMA,
            pltpu.SemaphoreType.DMA,
            pltpu.SemaphoreType.DMA,
        ],
    )(_edge_body)


def _vgather(vec, idx):
    dn = lax.GatherDimensionNumbers(offset_dims=(), collapsed_slice_dims=(0,),
                                    start_index_map=(0,))
    return lax.gather(vec, idx[:, None], dn, slice_sizes=(1,),
                      mode=lax.GatherScatterMode.PROMISE_IN_BOUNDS)


def _edge_body(ei_hbm, el_hbm, er_hbm, g_hbm, out_hbm,
               el_v, er_v, src0, src1, dst0, dst1, sci0, sci1,
               gbuf0, gbuf1, wbuf,
               out_sh, el_sh, er_sh,
               sem_sd0, sem_sd1, sem_g0, sem_g1, sem_sc0, sem_sc1):
    cid = lax.axis_index("c")
    sid = lax.axis_index("s")
    wid = cid * NS + sid

    srcs, dsts, scis = (src0, src1), (dst0, dst1), (sci0, sci1)
    gbufs = (gbuf0, gbuf1)
    sem_sd, sem_g, sem_sc = (sem_sd0, sem_sd1), (sem_g0, sem_g1), (sem_sc0,
                                                                   sem_sc1)

    lanes = lax.iota(jnp.int32, L)
    zero16 = jnp.zeros((L,), jnp.float32)

    def _zero_g(e, _):
        for pch in range(GW // L):
            gbuf0[e, pl.ds(pch * L, L)] = zero16
        return 0

    lax.fori_loop(0, WIN, _zero_g, 0)

    r0 = sid * STRIPE
    for k in range(STRIPE // WIN):
        pltpu.sync_copy(gbuf0, out_sh.at[pl.ds(r0 + k * WIN, WIN)])
    rem = STRIPE - (STRIPE // WIN) * WIN
    pltpu.sync_copy(gbuf0.at[pl.ds(0, rem)],
                    out_sh.at[pl.ds(r0 + STRIPE - rem, rem)])
    tail0 = NS * STRIPE
    tail = N_NODES - tail0

    @pl.when(sid == NS - 1)
    def _():
        pltpu.sync_copy(gbuf0.at[pl.ds(0, tail)], out_sh.at[pl.ds(tail0, tail)])

    @pl.when(sid == 0)
    def _():
        pltpu.sync_copy(el_hbm, el_sh)

    @pl.when(sid == 1)
    def _():
        pltpu.sync_copy(er_hbm, er_sh)

    plsc.subcore_barrier()
    pltpu.sync_copy(el_sh, el_v)
    pltpu.sync_copy(er_sh, er_v)

    hmaps = []
    for pch in range(GW // L):
        c = lanes + L * pch
        hmaps.append((c >= HP).astype(jnp.int32)
                     + (c >= 2 * HP).astype(jnp.int32)
                     + (c >= 3 * HP).astype(jnp.int32))

    def start_sd(w, ph):
        base = wid * EW + w * WIN
        pltpu.async_copy(ei_hbm.at[0, pl.ds(base, WIN)], srcs[ph], sem_sd[ph])
        pltpu.async_copy(ei_hbm.at[1, pl.ds(base, WIN)], dsts[ph], sem_sd[ph])

    def wait_sd(ph):
        pltpu.make_async_copy(ei_hbm.at[0, pl.ds(0, WIN)], srcs[ph],
                              sem_sd[ph]).wait()
        pltpu.make_async_copy(ei_hbm.at[1, pl.ds(0, WIN)], dsts[ph],
                              sem_sd[ph]).wait()

    def compute_w(ph):
        h4 = lanes & 3
        e4base = lanes >> 2
        for q in range(WIN * N_HEADS // L):
            e4 = e4base + q * 4
            srcr = plsc.load_gather(srcs[ph], [e4])
            dstr = plsc.load_gather(dsts[ph], [e4])
            e = (plsc.load_gather(el_v, [srcr * N_HEADS + h4])
                 + plsc.load_gather(er_v, [dstr * N_HEADS + h4]))
            e = jnp.maximum(e, 0.2 * e)
            wbuf[pl.ds(q * L, L)] = jnp.exp(e)

    def copy_sci(ph):
        for q in range(WIN // L):
            scis[ph][pl.ds(q * L, L)] = dsts[ph][pl.ds(q * L, L)]

    def mul(ph):
        for q in range(WIN * N_HEADS // L):
            w16 = wbuf[pl.ds(q * L, L)]
            for j in range(4):
                e = q * 4 + j
                for pch in range(GW // L):
                    wb = _vgather(w16, hmaps[pch] + 4 * j)
                    gv = gbufs[ph][e, pl.ds(pch * L, L)]
                    gbufs[ph][e, pl.ds(pch * L, L)] = gv * wb

    def phase(w, ph):
        wait_sd(ph)

        @pl.when(w >= 2)
        def _():
            pltpu.make_async_copy(gbufs[ph], out_sh.at[scis[ph]],
                                  sem_sc[ph]).wait()

        gdesc = pltpu.async_copy(g_hbm.at[srcs[ph]], gbufs[ph], sem_g[ph])
        compute_w(ph)
        copy_sci(ph)
        gdesc.wait()

        @pl.when(w + 2 < NWIN)
        def _():
            start_sd(w + 2, ph)

        mul(ph)
        pltpu.async_copy(gbufs[ph], out_sh.at[scis[ph]], sem_sc[ph], add=True)

    start_sd(jnp.int32(0), 0)
    start_sd(jnp.int32(1), 1)

    def _pair(i, _):
        w = i * 2
        phase(w, 0)

        @pl.when(w + 1 < NWIN)
        def _():
            phase(w + 1, 1)

        return 0

    lax.fori_loop(0, (NWIN + 1) // 2, _pair, 0)

    for ph in range(2):
        pltpu.make_async_copy(gbufs[ph], out_sh.at[scis[ph]],
                              sem_sc[ph]).wait()

    plsc.subcore_barrier()
    pltpu.sync_copy(out_sh.at[pl.ds(r0, STRIPE)],
                    out_hbm.at[cid, pl.ds(r0, STRIPE)])

    @pl.when(sid == NS - 1)
    def _():
        pltpu.sync_copy(out_sh.at[pl.ds(tail0, tail)],
                        out_hbm.at[cid, pl.ds(tail0, tail)])


def _combine_body(op_ref, bias_ref, fcw_ref, fcb_ref, o_ref):
    num = op_ref[0] + op_ref[1]
    const = lax.dot_general(bias_ref[...], fcw_ref[...],
                            (((1,), (1,)), ((), ())),
                            preferred_element_type=jnp.float32)
    const = const + fcb_ref[...]
    for h in range(N_HEADS):
        den = num[:, h * HP + FC_OUT:h * HP + FC_OUT + 1] + 1e-16
        o_ref[:, h * FC_OUT:(h + 1) * FC_OUT] = (
            num[:, h * HP:h * HP + FC_OUT] / den + const[h][None, :])


def _combine(out_p, bias, fc_W, fc_b):
    return pl.pallas_call(
        _combine_body,
        grid=(GRID,),
        in_specs=[
            pl.BlockSpec((NC, BR, GW), lambda i: (0, i, 0)),
            pl.BlockSpec((N_HEADS, IN_F), lambda i: (0, 0)),
            pl.BlockSpec((FC_OUT, IN_F), lambda i: (0, 0)),
            pl.BlockSpec((1, FC_OUT), lambda i: (0, 0)),
        ],
        out_specs=pl.BlockSpec((BR, N_HEADS * FC_OUT), lambda i: (i, 0)),
        out_shape=jax.ShapeDtypeStruct((N_NODES, N_HEADS * FC_OUT),
                                       jnp.float32),
    )(out_p, bias, fc_W, fc_b)


def kernel(features, edge_index, W, attn_l, attn_r, bias, fc_W, fc_b):
    ei = edge_index.astype(jnp.int32)
    el, er, g = _project(features, W, attn_l, attn_r, fc_W)
    out_p = _build_edge_kernel()(ei, el.reshape(-1), er.reshape(-1), g)
    out = _combine(out_p, bias, fc_W, fc_b.reshape(1, FC_OUT))
    return out.reshape(N_NODES, N_HEADS, FC_OUT)


# speedup vs baseline: 185.7668x; 1.0020x over previous
import functools

import jax
import jax.numpy as jnp
from jax import lax
from jax.experimental import pallas as pl
from jax.experimental.pallas import tpu as pltpu
from jax.experimental.pallas import tpu_sc as plsc

N_NODES = 10000
N_EDGES = 320000
IN_F = 128
N_HEADS = 4
FC_OUT = 10
HP = 12
GW = N_HEADS * HP
YW = 64

NC, NS, L = 2, 16, 16
NW = NC * NS
EW = N_EDGES // NW
WIN = 80
NWIN = EW // WIN
STRIPE = 624

BR = 2000
GRID = N_NODES // BR


def _proj_body(x_ref, w_ref, al_ref, ar_ref, fcw_ref,
               el_ref, er_ref, g_ref, c_scr):
    @pl.when(pl.program_id(0) == 0)
    def _():
        cols = []
        for vec_ref in (al_ref, ar_ref):
            hs = []
            for h in range(N_HEADS):
                v = lax.dot_general(vec_ref[h], w_ref[h],
                                    (((0,), (0,)), ((), ())),
                                    preferred_element_type=jnp.float32)
                hs.append(v[:, None])
            cols.append(jnp.concatenate(hs, axis=1))
        pad2 = jnp.zeros((IN_F, HP - FC_OUT), jnp.float32)
        for h in range(N_HEADS):
            gh = lax.dot_general(w_ref[h], fcw_ref[...],
                                 (((0,), (1,)), ((), ())),
                                 preferred_element_type=jnp.float32)
            cols.append(jnp.concatenate([gh, pad2], axis=1))
        c_scr[...] = jnp.concatenate(cols, axis=1)

    y = jnp.dot(x_ref[...], c_scr[...], preferred_element_type=jnp.float32)
    el_ref[...] = y[:, 0:N_HEADS]
    er_ref[...] = y[:, N_HEADS:2 * N_HEADS]
    cg = 2 * N_HEADS
    colid = lax.broadcasted_iota(jnp.int32, (1, GW), 1)
    g_ref[...] = jnp.where(colid % HP == FC_OUT, 1.0, y[:, cg:cg + GW])


def _project(x, W, attn_l, attn_r, fc_W):
    return pl.pallas_call(
        _proj_body,
        grid=(GRID,),
        in_specs=[
            pl.BlockSpec((BR, IN_F), lambda i: (i, 0)),
            pl.BlockSpec((N_HEADS, IN_F, IN_F), lambda i: (0, 0, 0)),
            pl.BlockSpec((N_HEADS, IN_F), lambda i: (0, 0)),
            pl.BlockSpec((N_HEADS, IN_F), lambda i: (0, 0)),
            pl.BlockSpec((FC_OUT, IN_F), lambda i: (0, 0)),
        ],
        out_specs=[
            pl.BlockSpec((BR, N_HEADS), lambda i: (i, 0)),
            pl.BlockSpec((BR, N_HEADS), lambda i: (i, 0)),
            pl.BlockSpec((BR, GW), lambda i: (i, 0)),
        ],
        out_shape=[
            jax.ShapeDtypeStruct((N_NODES, N_HEADS), jnp.float32),
            jax.ShapeDtypeStruct((N_NODES, N_HEADS), jnp.float32),
            jax.ShapeDtypeStruct((N_NODES, GW), jnp.float32),
        ],
        scratch_shapes=[pltpu.VMEM((IN_F, YW - 2 * N_HEADS), jnp.float32)],
    )(x, W, attn_l, attn_r, fc_W)


@functools.lru_cache(maxsize=1)
def _build_edge_kernel():
    mesh = plsc.VectorSubcoreMesh(core_axis_name="c", subcore_axis_name="s",
                                  num_cores=NC, num_subcores=NS)
    return functools.partial(
        pl.kernel,
        out_type=jax.ShapeDtypeStruct((NC, N_NODES, GW), jnp.float32),
        mesh=mesh,
        compiler_params=pltpu.CompilerParams(needs_layout_passes=False,
                                             use_tc_tiling_on_sc=False),
        scratch_types=[
            pltpu.VMEM((N_NODES * N_HEADS,), jnp.float32),
            pltpu.VMEM((N_NODES * N_HEADS,), jnp.float32),
            pltpu.VMEM((WIN,), jnp.int32),
            pltpu.VMEM((WIN,), jnp.int32),
            pltpu.VMEM((WIN,), jnp.int32),
            pltpu.VMEM((WIN,), jnp.int32),
            pltpu.VMEM((WIN,), jnp.int32),
            pltpu.VMEM((WIN,), jnp.int32),
            pltpu.VMEM((WIN, GW), jnp.float32),
            pltpu.VMEM((WIN, GW), jnp.float32),
            pltpu.VMEM((WIN * N_HEADS,), jnp.float32),
            pltpu.VMEM_SHARED((N_NODES, GW), jnp.float32),
            pltpu.VMEM_SHARED((N_NODES * N_HEADS,), jnp.float32),
            pltpu.VMEM_SHARED((N_NODES * N_HEADS,), jnp.float32),
            pltpu.SemaphoreType.DMA,
            pltpu.SemaphoreType.DMA,
            pltpu.SemaphoreType.DMA,
            pltpu.SemaphoreType.DMA,
            pltpu.SemaphoreType.DMA,
            pltpu.SemaphoreType.DMA,
        ],
    )(_edge_body)


def _vgather(vec, idx):
    dn = lax.GatherDimensionNumbers(offset_dims=(), collapsed_slice_dims=(0,),
                                    start_index_map=(0,))
    return lax.gather(vec, idx[:, None], dn, slice_sizes=(1,),
                      mode=lax.GatherScatterMode.PROMISE_IN_BOUNDS)


def _edge_body(ei_hbm, el_hbm, er_hbm, g_hbm, out_hbm,
               el_v, er_v, src0, src1, dst0, dst1, sci0, sci1,
               gbuf0, gbuf1, wbuf,
               out_sh, el_sh, er_sh,
               sem_sd0, sem_sd1, sem_g0, sem_g1, sem_sc0, sem_sc1):
    cid = lax.axis_index("c")
    sid = lax.axis_index("s")
    wid = cid * NS + sid

    srcs, dsts, scis = (src0, src1), (dst0, dst1), (sci0, sci1)
    gbufs = (gbuf0, gbuf1)
    sem_sd, sem_g, sem_sc = (sem_sd0, sem_sd1), (sem_g0, sem_g1), (sem_sc0,
                                                                   sem_sc1)

    lanes = lax.iota(jnp.int32, L)
    zero16 = jnp.zeros((L,), jnp.float32)

    def _zero_g(e, _):
        for pch in range(GW // L):
            gbuf0[e, pl.ds(pch * L, L)] = zero16
        return 0

    lax.fori_loop(0, WIN, _zero_g, 0)

    r0 = sid * STRIPE
    for k in range(STRIPE // WIN):
        pltpu.sync_copy(gbuf0, out_sh.at[pl.ds(r0 + k * WIN, WIN)])
    rem = STRIPE - (STRIPE // WIN) * WIN
    pltpu.sync_copy(gbuf0.at[pl.ds(0, rem)],
                    out_sh.at[pl.ds(r0 + STRIPE - rem, rem)])
    tail0 = NS * STRIPE
    tail = N_NODES - tail0

    @pl.when(sid == NS - 1)
    def _():
        pltpu.sync_copy(gbuf0.at[pl.ds(0, tail)], out_sh.at[pl.ds(tail0, tail)])

    @pl.when(sid == 0)
    def _():
        pltpu.sync_copy(el_hbm, el_sh)

    @pl.when(sid == 1)
    def _():
        pltpu.sync_copy(er_hbm, er_sh)

    plsc.subcore_barrier()
    pltpu.sync_copy(el_sh, el_v)
    pltpu.sync_copy(er_sh, er_v)

    hmaps = []
    for pch in range(GW // L):
        c = lanes + L * pch
        hmaps.append((c >= HP).astype(jnp.int32)
                     + (c >= 2 * HP).astype(jnp.int32)
                     + (c >= 3 * HP).astype(jnp.int32))

    def start_sd(w, ph):
        base = wid * EW + w * WIN
        pltpu.async_copy(ei_hbm.at[0, pl.ds(base, WIN)], srcs[ph], sem_sd[ph])
        pltpu.async_copy(ei_hbm.at[1, pl.ds(base, WIN)], dsts[ph], sem_sd[ph])

    def wait_sd(ph):
        pltpu.make_async_copy(ei_hbm.at[0, pl.ds(0, WIN)], srcs[ph],
                              sem_sd[ph]).wait()
        pltpu.make_async_copy(ei_hbm.at[1, pl.ds(0, WIN)], dsts[ph],
                              sem_sd[ph]).wait()

    def compute_w(ph):
        sidx = lanes * N_HEADS
        for q in range(WIN // L):
            src16 = srcs[ph][pl.ds(q * L, L)] * N_HEADS
            dst16 = dsts[ph][pl.ds(q * L, L)] * N_HEADS
            for h in range(N_HEADS):
                e = (plsc.load_gather(el_v, [src16 + h])
                     + plsc.load_gather(er_v, [dst16 + h]))
                e = jnp.maximum(e, 0.2 * e)
                plsc.store_scatter(wbuf, [sidx + (q * L * N_HEADS + h)],
                                   jnp.exp(e))

    def copy_sci(ph):
        for q in range(WIN // L):
            scis[ph][pl.ds(q * L, L)] = dsts[ph][pl.ds(q * L, L)]

    def mul(ph):
        for q in range(WIN * N_HEADS // L):
            w16 = wbuf[pl.ds(q * L, L)]
            for j in range(4):
                e = q * 4 + j
                for pch in range(GW // L):
                    wb = _vgather(w16, hmaps[pch] + 4 * j)
                    gv = gbufs[ph][e, pl.ds(pch * L, L)]
                    gbufs[ph][e, pl.ds(pch * L, L)] = gv * wb

    def phase(w, ph):
        wait_sd(ph)

        @pl.when(w >= 2)
        def _():
            pltpu.make_async_copy(gbufs[ph], out_sh.at[scis[ph]],
                                  sem_sc[ph]).wait()

        gdesc = pltpu.async_copy(g_hbm.at[srcs[ph]], gbufs[ph], sem_g[ph])
        compute_w(ph)
        copy_sci(ph)
        gdesc.wait()

        @pl.when(w + 2 < NWIN)
        def _():
            start_sd(w + 2, ph)

        mul(ph)
        pltpu.async_copy(gbufs[ph], out_sh.at[scis[ph]], sem_sc[ph], add=True)

    start_sd(jnp.int32(0), 0)
    start_sd(jnp.int32(1), 1)

    def _pair(i, _):
        w = i * 2
        phase(w, 0)

        @pl.when(w + 1 < NWIN)
        def _():
            phase(w + 1, 1)

        return 0

    lax.fori_loop(0, (NWIN + 1) // 2, _pair, 0)

    for ph in range(2):
        pltpu.make_async_copy(gbufs[ph], out_sh.at[scis[ph]],
                              sem_sc[ph]).wait()

    plsc.subcore_barrier()
    pltpu.sync_copy(out_sh.at[pl.ds(r0, STRIPE)],
                    out_hbm.at[cid, pl.ds(r0, STRIPE)])

    @pl.when(sid == NS - 1)
    def _():
        pltpu.sync_copy(out_sh.at[pl.ds(tail0, tail)],
                        out_hbm.at[cid, pl.ds(tail0, tail)])


def _combine_body(op_ref, bias_ref, fcw_ref, fcb_ref, o_ref):
    num = op_ref[0] + op_ref[1]
    const = lax.dot_general(bias_ref[...], fcw_ref[...],
                            (((1,), (1,)), ((), ())),
                            preferred_element_type=jnp.float32)
    const = const + fcb_ref[...]
    for h in range(N_HEADS):
        den = num[:, h * HP + FC_OUT:h * HP + FC_OUT + 1] + 1e-16
        o_ref[:, h * FC_OUT:(h + 1) * FC_OUT] = (
            num[:, h * HP:h * HP + FC_OUT] / den + const[h][None, :])


def _combine(out_p, bias, fc_W, fc_b):
    return pl.pallas_call(
        _combine_body,
        grid=(GRID,),
        in_specs=[
            pl.BlockSpec((NC, BR, GW), lambda i: (0, i, 0)),
            pl.BlockSpec((N_HEADS, IN_F), lambda i: (0, 0)),
            pl.BlockSpec((FC_OUT, IN_F), lambda i: (0, 0)),
            pl.BlockSpec((1, FC_OUT), lambda i: (0, 0)),
        ],
        out_specs=pl.BlockSpec((BR, N_HEADS * FC_OUT), lambda i: (i, 0)),
        out_shape=jax.ShapeDtypeStruct((N_NODES, N_HEADS * FC_OUT),
                                       jnp.float32),
    )(out_p, bias, fc_W, fc_b)


def kernel(features, edge_index, W, attn_l, attn_r, bias, fc_W, fc_b):
    ei = edge_index.astype(jnp.int32)
    el, er, g = _project(features, W, attn_l, attn_r, fc_W)
    out_p = _build_edge_kernel()(ei, el.reshape(-1), er.reshape(-1), g)
    out = _combine(out_p, bias, fc_W, fc_b.reshape(1, FC_OUT))
    return out.reshape(N_NODES, N_HEADS, FC_OUT)
